# Initial kernel scaffold; baseline (speedup 1.0000x reference)
import jax
import jax.numpy as jnp
from jax.experimental import pallas as pl


def kernel(x_user, x_item, edge_index_ui, edge_index_iu, target_edge_index, Wl1_ui, bl1_ui, Wr1_ui, Wl1_iu, bl1_iu, Wr1_iu, Wl2_ui, bl2_ui, Wr2_ui, Wl2_iu, bl2_iu, Wr2_iu, Wc1, bc1, Wc2, bc2):
    raise NotImplementedError("write your pallas kernel here")


# speedup vs baseline: 1.7178x; 1.7178x over previous
import functools

import jax
import jax.numpy as jnp
from jax import lax
from jax.experimental import pallas as pl
from jax.experimental.pallas import tpu as pltpu
from jax.experimental.pallas import tpu_sc as plsc

N = 10000
D = 128
E = 320000
NC = 2
NS = 16
NW = NC * NS
EW = E // NW
ESC = E // NC
CH = 40
NCH = EW // CH
ZR = 160
NP = 10240
RT = NP // NS
CW = 16

_F32 = jnp.float32


def _zero_vmem_2d(ref, rows, width):
    z16 = jnp.zeros((16,), _F32)

    def body(r, _):
        for j in range(width // 16):
            ref[r, pl.ds(j * 16, 16)] = z16
        return 0

    lax.fori_loop(0, rows, body, 0, unroll=False)


def _make_agg_kernel(with_counts):
    mesh = plsc.VectorSubcoreMesh(core_axis_name="c", subcore_axis_name="s")
    n_out = 4 if with_counts else 2
    out_type = [jax.ShapeDtypeStruct((NC, NP, D), _F32) for _ in range(n_out)]
    scratch = [
        pltpu.VMEM_SHARED((NP, D), _F32),
        pltpu.VMEM((ZR, D), _F32),
        pltpu.VMEM((CH,), jnp.int32),
        pltpu.VMEM((CH,), jnp.int32),
        pltpu.VMEM((CH, D), _F32),
        pltpu.SemaphoreType.DMA,
    ]
    if with_counts:
        scratch.append(pltpu.VMEM((CH, D), _F32))

    def body(src_ui, src_iu, row_ui, col_ui, row_iu, col_iu, *refs):
        outs = refs[:n_out]
        acc, zbuf, sidx, didx, rows, sem = refs[n_out:n_out + 6]
        ones = refs[n_out + 6] if with_counts else None
        c = lax.axis_index("c")
        s = lax.axis_index("s")
        r0 = s * RT
        e0 = c * ESC + s * EW

        if with_counts:
            one16 = jnp.ones((16,), _F32)

            def ones_body(r, _):
                for j in range(D // 16):
                    ones[r, pl.ds(j * 16, 16)] = one16
                return 0

            lax.fori_loop(0, CH, ones_body, 0, unroll=False)

        def phase(src, row_h, col_h, out):
            _zero_vmem_2d(zbuf, ZR, D)
            for q in range(RT // ZR):
                pltpu.sync_copy(zbuf, acc.at[pl.ds(r0 + q * ZR, ZR)])
            plsc.subcore_barrier()

            def chunk(k, _):
                b = e0 + k * CH
                pltpu.sync_copy(col_h.at[pl.ds(b, CH)], didx)
                if src is None:
                    pltpu.sync_copy(ones, acc.at[didx], add=True)
                else:
                    pltpu.sync_copy(row_h.at[pl.ds(b, CH)], sidx)
                    pltpu.async_copy(src.at[sidx], rows, sem).wait()
                    pltpu.sync_copy(rows, acc.at[didx], add=True)
                return 0

            lax.fori_loop(0, NCH, chunk, 0, unroll=False)
            plsc.subcore_barrier()
            for q in range(RT // ZR):
                pltpu.sync_copy(acc.at[pl.ds(r0 + q * ZR, ZR)], zbuf)
                pltpu.sync_copy(zbuf, out.at[c, pl.ds(r0 + q * ZR, ZR)])
            plsc.subcore_barrier()

        phase(src_ui, row_ui, col_ui, outs[0])
        phase(src_iu, row_iu, col_iu, outs[1])
        if with_counts:
            phase(None, None, col_ui, outs[2])
            phase(None, None, col_iu, outs[3])

    return pl.kernel(body, out_type=out_type, mesh=mesh,
                     scratch_types=scratch)


def _classifier_kernel():
    mesh = plsc.VectorSubcoreMesh(core_axis_name="c", subcore_axis_name="s")
    out_type = jax.ShapeDtypeStruct((E,), _F32)
    scratch = [
        pltpu.VMEM((CH,), jnp.int32),
        pltpu.VMEM((CH,), jnp.int32),
        pltpu.VMEM((CH, D), _F32),
        pltpu.VMEM((CH, D), _F32),
        pltpu.VMEM((CH,), _F32),
        pltpu.VMEM((D,), _F32),
        pltpu.VMEM((D,), _F32),
        pltpu.VMEM((16,), _F32),
        pltpu.SemaphoreType.DMA,
    ]

    def body(zu_h, zi_h, row_h, col_h, b1_h, w2_h, b2_h, out_h,
             ridx, cidx, zu, zi, ob, b1, w2, b2, sem):
        c = lax.axis_index("c")
        s = lax.axis_index("s")
        pltpu.sync_copy(b1_h, b1)
        pltpu.sync_copy(w2_h, w2)
        pltpu.sync_copy(b2_h, b2)
        bvs = [b1[pl.ds(j * 16, 16)] for j in range(D // 16)]
        wvs = [w2[pl.ds(j * 16, 16)] for j in range(D // 16)]
        b2v = b2[pl.ds(0, 16)]
        lanes = lax.iota(jnp.int32, 16)
        e0 = c * ESC + s * EW

        def chunk(k, _):
            b = e0 + k * CH
            pltpu.sync_copy(row_h.at[pl.ds(b, CH)], ridx)
            pltpu.sync_copy(col_h.at[pl.ds(b, CH)], cidx)
            pltpu.async_copy(zu_h.at[ridx], zu, sem).wait()
            pltpu.async_copy(zi_h.at[cidx], zi, sem).wait()

            for off in (0, 16, 24):
                res = jnp.zeros((16,), _F32)
                for t in range(16):
                    e = off + t
                    a = jnp.maximum(zu[e, pl.ds(0, 16)] + zi[e, pl.ds(0, 16)]
                                    + bvs[0], 0.0) * wvs[0]
                    for j in range(1, D // 16):
                        zuv = zu[e, pl.ds(j * 16, 16)]
                        ziv = zi[e, pl.ds(j * 16, 16)]
                        a = a + jnp.maximum(zuv + ziv + bvs[j], 0.0) * wvs[j]
                    res = jnp.where(lanes == t, jnp.sum(a), res)
                ob[pl.ds(off, 16)] = res + b2v
            pltpu.sync_copy(ob, out_h.at[pl.ds(b, CH)])
            return 0

        lax.fori_loop(0, NCH, chunk, 0, unroll=False)

    return pl.kernel(
        body, out_type=out_type, mesh=mesh, scratch_types=scratch,
        compiler_params=pltpu.CompilerParams(needs_layout_passes=False))


def _dot(a, b):
    return jnp.dot(a, b, precision=jax.lax.Precision.HIGHEST,
                   preferred_element_type=_F32)


_BS = 1000


def _sage_layer_kernel(pui, cui, piu, ciu, xd_ui, xd_iu,
                       Wl_ui, bl_ui, Wr_ui, Wl_iu, bl_iu, Wr_iu,
                       relu, proj_ui=None, proj_iu=None):
    with_proj = proj_ui is not None

    def tc_body(pui_r, cui_r, piu_r, ciu_r, xu_r, xi_r,
                wlu_r, blu_r, wru_r, wli_r, bli_r, wri_r, *rest):
        if with_proj:
            pu_r, pi_r, out_ui_r, out_iu_r = rest
        else:
            out_ui_r, out_iu_r = rest

        def side(p_r, c_r, x_r, wl_r, bl_r, wr_r):
            ssum = p_r[0] + p_r[1]
            cnt = jnp.maximum(c_r[0, :, 0:1] + c_r[1, :, 0:1], 1.0)
            h = _dot(ssum / cnt, wl_r[...]) + bl_r[...] + _dot(x_r[...], wr_r[...])
            return jnp.maximum(h, 0.0) if relu else h

        h_ui = side(pui_r, cui_r, xi_r, wlu_r, blu_r, wru_r)
        h_iu = side(piu_r, ciu_r, xu_r, wli_r, bli_r, wri_r)
        if with_proj:
            out_ui_r[...] = _dot(h_ui, pi_r[...])
            out_iu_r[...] = _dot(h_iu, pu_r[...])
        else:
            out_ui_r[...] = h_ui
            out_iu_r[...] = h_iu

    grid = (N // _BS,)
    part_spec = pl.BlockSpec((NC, _BS, D), lambda i: (0, i, 0))
    cnt_spec = pl.BlockSpec((NC, _BS, D), lambda i: (0, i, 0))
    row_spec = pl.BlockSpec((_BS, D), lambda i: (i, 0))
    w_spec = pl.BlockSpec((D, D), lambda i: (0, 0))
    b_spec = pl.BlockSpec((1, D), lambda i: (0, 0))
    in_specs = [part_spec, cnt_spec, part_spec, cnt_spec, row_spec, row_spec,
                w_spec, b_spec, w_spec, w_spec, b_spec, w_spec]
    args = [pui, cui, piu, ciu, xd_iu, xd_ui,
            Wl_ui, bl_ui.reshape(1, D), Wr_ui,
            Wl_iu, bl_iu.reshape(1, D), Wr_iu]
    if with_proj:
        in_specs += [w_spec, w_spec]
        args += [proj_ui, proj_iu]
    out_shape = [jax.ShapeDtypeStruct((N, D), _F32),
                 jax.ShapeDtypeStruct((N, D), _F32)]
    return pl.pallas_call(
        tc_body, grid=grid, in_specs=in_specs,
        out_specs=[row_spec, row_spec], out_shape=out_shape,
    )(*args)


def kernel(x_user, x_item, edge_index_ui, edge_index_iu, target_edge_index,
           Wl1_ui, bl1_ui, Wr1_ui, Wl1_iu, bl1_iu, Wr1_iu,
           Wl2_ui, bl2_ui, Wr2_ui, Wl2_iu, bl2_iu, Wr2_iu,
           Wc1, bc1, Wc2, bc2):
    x_user = x_user.astype(_F32)
    x_item = x_item.astype(_F32)
    row_ui = edge_index_ui[0].astype(jnp.int32)
    col_ui = edge_index_ui[1].astype(jnp.int32)
    row_iu = edge_index_iu[0].astype(jnp.int32)
    col_iu = edge_index_iu[1].astype(jnp.int32)
    row_t = target_edge_index[0].astype(jnp.int32)
    col_t = target_edge_index[1].astype(jnp.int32)

    agg1 = _make_agg_kernel(True)
    p_ui, p_iu, c_ui, c_iu = agg1(x_user, x_item, row_ui, col_ui, row_iu, col_iu)
    h_item, h_user = _sage_layer_kernel(
        p_ui, c_ui, p_iu, c_iu, x_item, x_user,
        Wl1_ui, bl1_ui, Wr1_ui, Wl1_iu, bl1_iu, Wr1_iu, relu=True)

    agg2 = _make_agg_kernel(False)
    q_ui, q_iu = agg2(h_user, h_item, row_ui, col_ui, row_iu, col_iu)
    ZI, ZU = _sage_layer_kernel(
        q_ui, c_ui, q_iu, c_iu, h_item, h_user,
        Wl2_ui, bl2_ui, Wr2_ui, Wl2_iu, bl2_iu, Wr2_iu, relu=False,
        proj_ui=Wc1[:D].astype(_F32), proj_iu=Wc1[D:].astype(_F32))

    clf = _classifier_kernel()
    out = clf(ZU, ZI, row_t, col_t, bc1.astype(_F32), Wc2[:, 0].astype(_F32),
              jnp.broadcast_to(bc2.astype(_F32), (16,)))
    return out
